# Initial kernel scaffold; baseline (speedup 1.0000x reference)
import jax
import jax.numpy as jnp
from jax.experimental import pallas as pl


def kernel(x, edge_index, batch, Wc0, Wa0, Wr0, b0, Wc1, Wa1, Wr1, b1, Wl, bl):
    raise NotImplementedError("write your pallas kernel here")


# speedup vs baseline: 5.1732x; 5.1732x over previous
import functools

import jax
import jax.numpy as jnp
from jax import lax
from jax.experimental import pallas as pl
from jax.experimental.pallas import tpu as pltpu
from jax.experimental.pallas import tpu_sc as plsc

NUM_SC = 2
NUM_TILES = 16
NUM_W = NUM_SC * NUM_TILES
EDGE_CHUNK = 80
NUM_GRAPHS = 16
BLK = 1000


def _sc_agg(x, src, dst):
    N, D = x.shape
    E = src.shape[0]
    e_per_w = E // NUM_W
    n_chunks = e_per_w // EDGE_CHUNK
    zrows = 80
    n_row_chunks = N // zrows
    row_rounds = -(-n_row_chunks // NUM_TILES)
    mesh = plsc.VectorSubcoreMesh(core_axis_name="c", subcore_axis_name="s")

    @functools.partial(
        pl.kernel,
        out_type=jax.ShapeDtypeStruct((NUM_SC, N, D), jnp.float32),
        mesh=mesh,
        scratch_types=[
            pltpu.VMEM((EDGE_CHUNK,), jnp.int32),
            pltpu.VMEM((EDGE_CHUNK,), jnp.int32),
            pltpu.VMEM((EDGE_CHUNK, D), jnp.float32),
            pltpu.VMEM((zrows, D), jnp.float32),
            pltpu.VMEM_SHARED((N, D), jnp.float32),
            pltpu.SemaphoreType.DMA,
        ],
    )
    def agg_kernel(x_hbm, src_hbm, dst_hbm, out_hbm, sidx, didx, rows, zbuf,
                   acc, sem):
        c = lax.axis_index("c")
        s = lax.axis_index("s")
        w = c * NUM_TILES + s

        def zero_body(i, carry):
            for j in range(D // 16):
                zbuf[i, pl.ds(j * 16, 16)] = jnp.zeros((16,), jnp.float32)
            return carry

        lax.fori_loop(0, zrows, zero_body, 0)
        for k in range(row_rounds):
            j = s + k * NUM_TILES

            @pl.when(j < n_row_chunks)
            def _():
                pltpu.sync_copy(
                    zbuf, acc.at[pl.ds(pl.multiple_of(j * zrows, 8), zrows)])
        plsc.subcore_barrier()

        def edge_body(i, carry):
            base = pl.multiple_of(w * e_per_w + i * EDGE_CHUNK, 8)
            pltpu.sync_copy(src_hbm.at[pl.ds(base, EDGE_CHUNK)], sidx)
            pltpu.sync_copy(dst_hbm.at[pl.ds(base, EDGE_CHUNK)], didx)
            pltpu.async_copy(x_hbm.at[sidx], rows, sem).wait()
            pltpu.sync_copy(rows, acc.at[didx], add=True)
            return carry

        lax.fori_loop(0, n_chunks, edge_body, 0)
        plsc.subcore_barrier()

        for k in range(row_rounds):
            j = s + k * NUM_TILES

            @pl.when(j < n_row_chunks)
            def _():
                base = pl.multiple_of(j * zrows, 8)
                pltpu.sync_copy(acc.at[pl.ds(base, zrows)],
                                out_hbm.at[c, pl.ds(base, zrows)])

    return agg_kernel(x, src, dst)


def _onehot_t(batch_ref, blk):
    bb = jnp.broadcast_to(batch_ref[0], (NUM_GRAPHS, blk))
    ids = lax.broadcasted_iota(jnp.int32, (NUM_GRAPHS, blk), 0)
    return (bb == ids).astype(jnp.float32)


def _tc_read(x, batch3d):
    N, D = x.shape

    def body(batch_ref, x_ref, read_ref):
        i = pl.program_id(0)
        part = jnp.dot(_onehot_t(batch_ref, BLK), x_ref[...],
                       preferred_element_type=jnp.float32)

        @pl.when(i == 0)
        def _():
            read_ref[...] = part

        @pl.when(i != 0)
        def _():
            read_ref[...] += part

    return pl.pallas_call(
        body,
        grid=(N // BLK,),
        in_specs=[
            pl.BlockSpec((1, 1, BLK), lambda i: (i, 0, 0)),
            pl.BlockSpec((BLK, D), lambda i: (i, 0)),
        ],
        out_specs=pl.BlockSpec((NUM_GRAPHS, D), lambda i: (0, 0)),
        out_shape=jax.ShapeDtypeStruct((NUM_GRAPHS, D), jnp.float32),
    )(batch3d, x)


def _layer_block(batch_ref, x_ref, a0_ref, a1_ref, read_ref, Wc_ref, Wa_ref,
                 Wr_ref, b_ref):
    r = jnp.dot(read_ref[...], Wr_ref[...], preferred_element_type=jnp.float32)
    oht = _onehot_t(batch_ref, BLK)
    rb = lax.dot_general(oht, r, (((0,), (0,)), ((), ())),
                         preferred_element_type=jnp.float32)
    agg = a0_ref[0] + a1_ref[0]
    h = jnp.dot(x_ref[...], Wc_ref[...], preferred_element_type=jnp.float32)
    h = h + jnp.dot(agg, Wa_ref[...], preferred_element_type=jnp.float32)
    h = h + rb + b_ref[...]
    return jnp.maximum(h, 0.0), oht


def _tc_combine_mid(x, aggp, batch3d, read, Wc, Wa, Wr, b2):
    N, D = x.shape

    def body(batch_ref, x_ref, a0_ref, a1_ref, read_ref, Wc_ref, Wa_ref,
             Wr_ref, b_ref, h_ref, rd_ref):
        i = pl.program_id(0)
        h, oht = _layer_block(batch_ref, x_ref, a0_ref, a1_ref, read_ref,
                              Wc_ref, Wa_ref, Wr_ref, b_ref)
        h_ref[...] = h
        part = jnp.dot(oht, h, preferred_element_type=jnp.float32)

        @pl.when(i == 0)
        def _():
            rd_ref[...] = part

        @pl.when(i != 0)
        def _():
            rd_ref[...] += part

    full = lambda shape: pl.BlockSpec(shape, lambda i: tuple(0 for _ in shape))
    return pl.pallas_call(
        body,
        grid=(N // BLK,),
        in_specs=[
            pl.BlockSpec((1, 1, BLK), lambda i: (i, 0, 0)),
            pl.BlockSpec((BLK, D), lambda i: (i, 0)),
            pl.BlockSpec((1, BLK, D), lambda i: (0, i, 0)),
            pl.BlockSpec((1, BLK, D), lambda i: (1, i, 0)),
            full((NUM_GRAPHS, D)),
            full((D, D)),
            full((D, D)),
            full((D, D)),
            full((1, D)),
        ],
        out_specs=[
            pl.BlockSpec((BLK, D), lambda i: (i, 0)),
            pl.BlockSpec((NUM_GRAPHS, D), lambda i: (0, 0)),
        ],
        out_shape=[
            jax.ShapeDtypeStruct((N, D), jnp.float32),
            jax.ShapeDtypeStruct((NUM_GRAPHS, D), jnp.float32),
        ],
    )(batch3d, x, aggp, aggp, read, Wc, Wa, Wr, b2)


def _tc_combine_final(x, aggp, batch3d, read, Wc, Wa, Wr, b2, Wlp, blp):
    N, D = x.shape

    def body(batch_ref, x_ref, a0_ref, a1_ref, read_ref, Wc_ref, Wa_ref,
             Wr_ref, b_ref, Wl_ref, bl_ref, o_ref):
        h, _ = _layer_block(batch_ref, x_ref, a0_ref, a1_ref, read_ref,
                            Wc_ref, Wa_ref, Wr_ref, b_ref)
        o_ref[...] = jnp.dot(h, Wl_ref[...],
                             preferred_element_type=jnp.float32) + bl_ref[...]

    full = lambda shape: pl.BlockSpec(shape, lambda i: tuple(0 for _ in shape))
    return pl.pallas_call(
        body,
        grid=(N // BLK,),
        in_specs=[
            pl.BlockSpec((1, 1, BLK), lambda i: (i, 0, 0)),
            pl.BlockSpec((BLK, D), lambda i: (i, 0)),
            pl.BlockSpec((1, BLK, D), lambda i: (0, i, 0)),
            pl.BlockSpec((1, BLK, D), lambda i: (1, i, 0)),
            full((NUM_GRAPHS, D)),
            full((D, D)),
            full((D, D)),
            full((D, D)),
            full((1, D)),
            full((D, 128)),
            full((1, 128)),
        ],
        out_specs=pl.BlockSpec((BLK, 128), lambda i: (i, 0)),
        out_shape=jax.ShapeDtypeStruct((N, 128), jnp.float32),
    )(batch3d, x, aggp, aggp, read, Wc, Wa, Wr, b2, Wlp, blp)


def kernel(x, edge_index, batch, Wc0, Wa0, Wr0, b0, Wc1, Wa1, Wr1, b1, Wl, bl):
    N, D = x.shape
    C = Wl.shape[1]
    src = edge_index[0]
    dst = edge_index[1]
    batch3d = batch.reshape(N // BLK, 1, BLK)
    b0r = b0.reshape(1, D)
    b1r = b1.reshape(1, D)
    Wlp = jnp.zeros((D, 128), jnp.float32).at[:, :C].set(Wl)
    blp = jnp.zeros((1, 128), jnp.float32).at[0, :C].set(bl)

    read0 = _tc_read(x, batch3d)
    aggp0 = _sc_agg(x, src, dst)
    h0, read1 = _tc_combine_mid(x, aggp0, batch3d, read0, Wc0, Wa0, Wr0, b0r)
    aggp1 = _sc_agg(h0, src, dst)
    outp = _tc_combine_final(h0, aggp1, batch3d, read1, Wc1, Wa1, Wr1, b1r,
                             Wlp, blp)
    return outp[:, :C]


# speedup vs baseline: 12.0962x; 2.3382x over previous
import functools

import jax
import jax.numpy as jnp
from jax import lax
from jax.experimental import pallas as pl
from jax.experimental.pallas import tpu as pltpu
from jax.experimental.pallas import tpu_sc as plsc

NUM_SC = 2
NUM_TILES = 16
NUM_W = NUM_SC * NUM_TILES
EDGE_CHUNK = 100
NUM_GRAPHS = 16
BLK = 1000


def _sc_agg(x, src3, dst3):
    N, D = x.shape
    _, n_halves, hchunks, _ = src3.shape
    zrows = 40
    n_zero_chunks = N // zrows
    zero_rounds = -(-n_zero_chunks // NUM_TILES)
    orows = 80
    n_row_chunks = N // orows
    row_rounds = -(-n_row_chunks // NUM_TILES)
    mesh = plsc.VectorSubcoreMesh(core_axis_name="c", subcore_axis_name="s")

    @functools.partial(
        pl.kernel,
        out_type=jax.ShapeDtypeStruct((NUM_SC, N, D), jnp.float32),
        mesh=mesh,
        scratch_types=[
            pltpu.VMEM((hchunks, EDGE_CHUNK), jnp.int32),
            pltpu.VMEM((hchunks, EDGE_CHUNK), jnp.int32),
            pltpu.VMEM((EDGE_CHUNK, D), jnp.float32),
            pltpu.VMEM((EDGE_CHUNK, D), jnp.float32),
            pltpu.VMEM((zrows, D), jnp.float32),
            pltpu.VMEM_SHARED((N, D), jnp.float32),
            pltpu.SemaphoreType.DMA,
            pltpu.SemaphoreType.DMA,
        ],
    )
    def agg_kernel(x_hbm, src_hbm, dst_hbm, out_hbm, sidx, didx, rows0, rows1,
                   zbuf, acc, gsem0, gsem1):
        c = lax.axis_index("c")
        s = lax.axis_index("s")
        w = c * NUM_TILES + s
        rows = (rows0, rows1)
        gsem = (gsem0, gsem1)

        def zero_body(i, carry):
            for j in range(D // 16):
                zbuf[i, pl.ds(j * 16, 16)] = jnp.zeros((16,), jnp.float32)
            return carry

        lax.fori_loop(0, zrows, zero_body, 0)
        for k in range(zero_rounds):
            j = s + k * NUM_TILES

            @pl.when(j < n_zero_chunks)
            def _():
                pltpu.sync_copy(
                    zbuf, acc.at[pl.ds(pl.multiple_of(j * zrows, 8), zrows)])
        plsc.subcore_barrier()

        for half in range(n_halves):
            pltpu.sync_copy(src_hbm.at[w, half], sidx)
            pltpu.sync_copy(dst_hbm.at[w, half], didx)
            pltpu.async_copy(x_hbm.at[sidx.at[0]], rows0, gsem0)
            pltpu.async_copy(x_hbm.at[sidx.at[1]], rows1, gsem1)

            def edge_body(k, carry):
                for b in range(2):
                    i = 2 * k + b
                    pltpu.make_async_copy(x_hbm.at[sidx.at[i]], rows[b],
                                          gsem[b]).wait()
                    pltpu.sync_copy(rows[b], acc.at[didx.at[i]], add=True)

                    @pl.when(i + 2 < hchunks)
                    def _():
                        pltpu.async_copy(x_hbm.at[sidx.at[i + 2]], rows[b],
                                         gsem[b])
                return carry

            lax.fori_loop(0, hchunks // 2, edge_body, 0)
        plsc.subcore_barrier()

        for k in range(row_rounds):
            j = s + k * NUM_TILES

            @pl.when(j < n_row_chunks)
            def _():
                base = pl.multiple_of(j * orows, 8)
                pltpu.sync_copy(acc.at[pl.ds(base, orows)],
                                out_hbm.at[c, pl.ds(base, orows)])

    return agg_kernel(x, src3, dst3)


def _onehot_t(batch_ref, blk):
    bb = jnp.broadcast_to(batch_ref[0], (NUM_GRAPHS, blk))
    ids = lax.broadcasted_iota(jnp.int32, (NUM_GRAPHS, blk), 0)
    return (bb == ids).astype(jnp.float32)


def _tc_read(x, batch3d):
    N, D = x.shape

    def body(batch_ref, x_ref, read_ref):
        i = pl.program_id(0)
        part = jnp.dot(_onehot_t(batch_ref, BLK), x_ref[...],
                       preferred_element_type=jnp.float32)

        @pl.when(i == 0)
        def _():
            read_ref[...] = part

        @pl.when(i != 0)
        def _():
            read_ref[...] += part

    return pl.pallas_call(
        body,
        grid=(N // BLK,),
        in_specs=[
            pl.BlockSpec((1, 1, BLK), lambda i: (i, 0, 0)),
            pl.BlockSpec((BLK, D), lambda i: (i, 0)),
        ],
        out_specs=pl.BlockSpec((NUM_GRAPHS, D), lambda i: (0, 0)),
        out_shape=jax.ShapeDtypeStruct((NUM_GRAPHS, D), jnp.float32),
    )(batch3d, x)


def _layer_block(batch_ref, x_ref, a0_ref, a1_ref, read_ref, Wc_ref, Wa_ref,
                 Wr_ref, b_ref):
    r = jnp.dot(read_ref[...], Wr_ref[...], preferred_element_type=jnp.float32)
    oht = _onehot_t(batch_ref, BLK)
    rb = lax.dot_general(oht, r, (((0,), (0,)), ((), ())),
                         preferred_element_type=jnp.float32)
    agg = a0_ref[0] + a1_ref[0]
    h = jnp.dot(x_ref[...], Wc_ref[...], preferred_element_type=jnp.float32)
    h = h + jnp.dot(agg, Wa_ref[...], preferred_element_type=jnp.float32)
    h = h + rb + b_ref[...]
    return jnp.maximum(h, 0.0), oht


def _tc_combine_mid(x, aggp, batch3d, read, Wc, Wa, Wr, b2):
    N, D = x.shape

    def body(batch_ref, x_ref, a0_ref, a1_ref, read_ref, Wc_ref, Wa_ref,
             Wr_ref, b_ref, h_ref, rd_ref):
        i = pl.program_id(0)
        h, oht = _layer_block(batch_ref, x_ref, a0_ref, a1_ref, read_ref,
                              Wc_ref, Wa_ref, Wr_ref, b_ref)
        h_ref[...] = h
        part = jnp.dot(oht, h, preferred_element_type=jnp.float32)

        @pl.when(i == 0)
        def _():
            rd_ref[...] = part

        @pl.when(i != 0)
        def _():
            rd_ref[...] += part

    full = lambda shape: pl.BlockSpec(shape, lambda i: tuple(0 for _ in shape))
    return pl.pallas_call(
        body,
        grid=(N // BLK,),
        in_specs=[
            pl.BlockSpec((1, 1, BLK), lambda i: (i, 0, 0)),
            pl.BlockSpec((BLK, D), lambda i: (i, 0)),
            pl.BlockSpec((1, BLK, D), lambda i: (0, i, 0)),
            pl.BlockSpec((1, BLK, D), lambda i: (1, i, 0)),
            full((NUM_GRAPHS, D)),
            full((D, D)),
            full((D, D)),
            full((D, D)),
            full((1, D)),
        ],
        out_specs=[
            pl.BlockSpec((BLK, D), lambda i: (i, 0)),
            pl.BlockSpec((NUM_GRAPHS, D), lambda i: (0, 0)),
        ],
        out_shape=[
            jax.ShapeDtypeStruct((N, D), jnp.float32),
            jax.ShapeDtypeStruct((NUM_GRAPHS, D), jnp.float32),
        ],
    )(batch3d, x, aggp, aggp, read, Wc, Wa, Wr, b2)


def _tc_combine_final(x, aggp, batch3d, read, Wc, Wa, Wr, b2, Wlp, blp):
    N, D = x.shape

    def body(batch_ref, x_ref, a0_ref, a1_ref, read_ref, Wc_ref, Wa_ref,
             Wr_ref, b_ref, Wl_ref, bl_ref, o_ref):
        h, _ = _layer_block(batch_ref, x_ref, a0_ref, a1_ref, read_ref,
                            Wc_ref, Wa_ref, Wr_ref, b_ref)
        o_ref[...] = jnp.dot(h, Wl_ref[...],
                             preferred_element_type=jnp.float32) + bl_ref[...]

    full = lambda shape: pl.BlockSpec(shape, lambda i: tuple(0 for _ in shape))
    return pl.pallas_call(
        body,
        grid=(N // BLK,),
        in_specs=[
            pl.BlockSpec((1, 1, BLK), lambda i: (i, 0, 0)),
            pl.BlockSpec((BLK, D), lambda i: (i, 0)),
            pl.BlockSpec((1, BLK, D), lambda i: (0, i, 0)),
            pl.BlockSpec((1, BLK, D), lambda i: (1, i, 0)),
            full((NUM_GRAPHS, D)),
            full((D, D)),
            full((D, D)),
            full((D, D)),
            full((1, D)),
            full((D, 128)),
            full((1, 128)),
        ],
        out_specs=pl.BlockSpec((BLK, 128), lambda i: (i, 0)),
        out_shape=jax.ShapeDtypeStruct((N, 128), jnp.float32),
    )(batch3d, x, aggp, aggp, read, Wc, Wa, Wr, b2, Wlp, blp)


def kernel(x, edge_index, batch, Wc0, Wa0, Wr0, b0, Wc1, Wa1, Wr1, b1, Wl, bl):
    N, D = x.shape
    C = Wl.shape[1]
    E = edge_index.shape[1]
    n_chunks = E // (NUM_W * EDGE_CHUNK)
    src3 = edge_index[0].reshape(NUM_W, 2, n_chunks // 2, EDGE_CHUNK)
    dst3 = edge_index[1].reshape(NUM_W, 2, n_chunks // 2, EDGE_CHUNK)
    batch3d = batch.reshape(N // BLK, 1, BLK)
    b0r = b0.reshape(1, D)
    b1r = b1.reshape(1, D)
    Wlp = jnp.zeros((D, 128), jnp.float32).at[:, :C].set(Wl)
    blp = jnp.zeros((1, 128), jnp.float32).at[0, :C].set(bl)

    read0 = _tc_read(x, batch3d)
    aggp0 = _sc_agg(x, src3, dst3)
    h0, read1 = _tc_combine_mid(x, aggp0, batch3d, read0, Wc0, Wa0, Wr0, b0r)
    aggp1 = _sc_agg(h0, src3, dst3)
    outp = _tc_combine_final(h0, aggp1, batch3d, read1, Wc1, Wa1, Wr1, b1r,
                             Wlp, blp)
    return outp[:, :C]


# speedup vs baseline: 12.4216x; 1.0269x over previous
import functools

import jax
import jax.numpy as jnp
from jax import lax
from jax.experimental import pallas as pl
from jax.experimental.pallas import tpu as pltpu
from jax.experimental.pallas import tpu_sc as plsc

NUM_SC = 2
NUM_TILES = 16
NUM_W = NUM_SC * NUM_TILES
EDGE_CHUNK = 100
NUM_GRAPHS = 16
BLK = 2000


def _sc_agg(x, src3, dst3):
    N, D = x.shape
    _, n_halves, hchunks, _ = src3.shape
    zrows = 40
    n_zero_chunks = N // zrows
    zero_rounds = -(-n_zero_chunks // NUM_TILES)
    orows = 80
    n_row_chunks = N // orows
    row_rounds = -(-n_row_chunks // NUM_TILES)
    mesh = plsc.VectorSubcoreMesh(core_axis_name="c", subcore_axis_name="s")

    @functools.partial(
        pl.kernel,
        out_type=jax.ShapeDtypeStruct((NUM_SC, N, D), jnp.float32),
        mesh=mesh,
        scratch_types=[
            pltpu.VMEM((hchunks, EDGE_CHUNK), jnp.int32),
            pltpu.VMEM((hchunks, EDGE_CHUNK), jnp.int32),
            pltpu.VMEM((EDGE_CHUNK, D), jnp.float32),
            pltpu.VMEM((EDGE_CHUNK, D), jnp.float32),
            pltpu.VMEM((zrows, D), jnp.float32),
            pltpu.VMEM_SHARED((N, D), jnp.float32),
            pltpu.SemaphoreType.DMA,
            pltpu.SemaphoreType.DMA,
        ],
    )
    def agg_kernel(x_hbm, src_hbm, dst_hbm, out_hbm, sidx, didx, rows0, rows1,
                   zbuf, acc, gsem0, gsem1):
        c = lax.axis_index("c")
        s = lax.axis_index("s")
        w = c * NUM_TILES + s
        rows = (rows0, rows1)
        gsem = (gsem0, gsem1)

        def zero_body(i, carry):
            for j in range(D // 16):
                zbuf[i, pl.ds(j * 16, 16)] = jnp.zeros((16,), jnp.float32)
            return carry

        lax.fori_loop(0, zrows, zero_body, 0)
        for k in range(zero_rounds):
            j = s + k * NUM_TILES

            @pl.when(j < n_zero_chunks)
            def _():
                pltpu.async_copy(
                    zbuf, acc.at[pl.ds(pl.multiple_of(j * zrows, 8), zrows)],
                    gsem0)
        for k in range(zero_rounds):
            j = s + k * NUM_TILES

            @pl.when(j < n_zero_chunks)
            def _():
                pltpu.make_async_copy(
                    zbuf, acc.at[pl.ds(pl.multiple_of(j * zrows, 8), zrows)],
                    gsem0).wait()
        plsc.subcore_barrier()

        for half in range(n_halves):
            pltpu.sync_copy(src_hbm.at[w, half], sidx)
            pltpu.sync_copy(dst_hbm.at[w, half], didx)
            pltpu.async_copy(x_hbm.at[sidx.at[0]], rows0, gsem0)
            pltpu.async_copy(x_hbm.at[sidx.at[1]], rows1, gsem1)

            def edge_body(k, carry):
                for b in range(2):
                    i = 2 * k + b
                    pltpu.make_async_copy(x_hbm.at[sidx.at[i]], rows[b],
                                          gsem[b]).wait()
                    pltpu.sync_copy(rows[b], acc.at[didx.at[i]], add=True)

                    @pl.when(i + 2 < hchunks)
                    def _():
                        pltpu.async_copy(x_hbm.at[sidx.at[i + 2]], rows[b],
                                         gsem[b])
                return carry

            lax.fori_loop(0, hchunks // 2, edge_body, 0)
        plsc.subcore_barrier()

        for k in range(row_rounds):
            j = s + k * NUM_TILES

            @pl.when(j < n_row_chunks)
            def _():
                base = pl.multiple_of(j * orows, 8)
                pltpu.async_copy(acc.at[pl.ds(base, orows)],
                                 out_hbm.at[c, pl.ds(base, orows)], gsem0)
        for k in range(row_rounds):
            j = s + k * NUM_TILES

            @pl.when(j < n_row_chunks)
            def _():
                base = pl.multiple_of(j * orows, 8)
                pltpu.make_async_copy(acc.at[pl.ds(base, orows)],
                                      out_hbm.at[c, pl.ds(base, orows)],
                                      gsem0).wait()

    return agg_kernel(x, src3, dst3)


def _onehot_t(batch_ref, blk):
    bb = jnp.broadcast_to(batch_ref[0], (NUM_GRAPHS, blk))
    ids = lax.broadcasted_iota(jnp.int32, (NUM_GRAPHS, blk), 0)
    return (bb == ids).astype(jnp.float32)


def _tc_read(x, batch3d):
    N, D = x.shape

    def body(batch_ref, x_ref, read_ref):
        i = pl.program_id(0)
        part = jnp.dot(_onehot_t(batch_ref, BLK), x_ref[...],
                       preferred_element_type=jnp.float32)

        @pl.when(i == 0)
        def _():
            read_ref[...] = part

        @pl.when(i != 0)
        def _():
            read_ref[...] += part

    return pl.pallas_call(
        body,
        grid=(N // BLK,),
        in_specs=[
            pl.BlockSpec((1, 1, BLK), lambda i: (i, 0, 0)),
            pl.BlockSpec((BLK, D), lambda i: (i, 0)),
        ],
        out_specs=pl.BlockSpec((NUM_GRAPHS, D), lambda i: (0, 0)),
        out_shape=jax.ShapeDtypeStruct((NUM_GRAPHS, D), jnp.float32),
    )(batch3d, x)


def _layer_block(batch_ref, x_ref, a0_ref, a1_ref, read_ref, Wc_ref, Wa_ref,
                 Wr_ref, b_ref):
    r = jnp.dot(read_ref[...], Wr_ref[...], preferred_element_type=jnp.float32)
    oht = _onehot_t(batch_ref, BLK)
    rb = lax.dot_general(oht, r, (((0,), (0,)), ((), ())),
                         preferred_element_type=jnp.float32)
    agg = a0_ref[0] + a1_ref[0]
    h = jnp.dot(x_ref[...], Wc_ref[...], preferred_element_type=jnp.float32)
    h = h + jnp.dot(agg, Wa_ref[...], preferred_element_type=jnp.float32)
    h = h + rb + b_ref[...]
    return jnp.maximum(h, 0.0), oht


def _tc_combine_mid(x, aggp, batch3d, read, Wc, Wa, Wr, b2):
    N, D = x.shape

    def body(batch_ref, x_ref, a0_ref, a1_ref, read_ref, Wc_ref, Wa_ref,
             Wr_ref, b_ref, h_ref, rd_ref):
        i = pl.program_id(0)
        h, oht = _layer_block(batch_ref, x_ref, a0_ref, a1_ref, read_ref,
                              Wc_ref, Wa_ref, Wr_ref, b_ref)
        h_ref[...] = h
        part = jnp.dot(oht, h, preferred_element_type=jnp.float32)

        @pl.when(i == 0)
        def _():
            rd_ref[...] = part

        @pl.when(i != 0)
        def _():
            rd_ref[...] += part

    full = lambda shape: pl.BlockSpec(shape, lambda i: tuple(0 for _ in shape))
    return pl.pallas_call(
        body,
        grid=(N // BLK,),
        in_specs=[
            pl.BlockSpec((1, 1, BLK), lambda i: (i, 0, 0)),
            pl.BlockSpec((BLK, D), lambda i: (i, 0)),
            pl.BlockSpec((1, BLK, D), lambda i: (0, i, 0)),
            pl.BlockSpec((1, BLK, D), lambda i: (1, i, 0)),
            full((NUM_GRAPHS, D)),
            full((D, D)),
            full((D, D)),
            full((D, D)),
            full((1, D)),
        ],
        out_specs=[
            pl.BlockSpec((BLK, D), lambda i: (i, 0)),
            pl.BlockSpec((NUM_GRAPHS, D), lambda i: (0, 0)),
        ],
        out_shape=[
            jax.ShapeDtypeStruct((N, D), jnp.float32),
            jax.ShapeDtypeStruct((NUM_GRAPHS, D), jnp.float32),
        ],
    )(batch3d, x, aggp, aggp, read, Wc, Wa, Wr, b2)


def _tc_combine_final(x, aggp, batch3d, read, Wc, Wa, Wr, b2, Wlp, blp):
    N, D = x.shape

    def body(batch_ref, x_ref, a0_ref, a1_ref, read_ref, Wc_ref, Wa_ref,
             Wr_ref, b_ref, Wl_ref, bl_ref, o_ref):
        h, _ = _layer_block(batch_ref, x_ref, a0_ref, a1_ref, read_ref,
                            Wc_ref, Wa_ref, Wr_ref, b_ref)
        o_ref[...] = jnp.dot(h, Wl_ref[...],
                             preferred_element_type=jnp.float32) + bl_ref[...]

    full = lambda shape: pl.BlockSpec(shape, lambda i: tuple(0 for _ in shape))
    return pl.pallas_call(
        body,
        grid=(N // BLK,),
        in_specs=[
            pl.BlockSpec((1, 1, BLK), lambda i: (i, 0, 0)),
            pl.BlockSpec((BLK, D), lambda i: (i, 0)),
            pl.BlockSpec((1, BLK, D), lambda i: (0, i, 0)),
            pl.BlockSpec((1, BLK, D), lambda i: (1, i, 0)),
            full((NUM_GRAPHS, D)),
            full((D, D)),
            full((D, D)),
            full((D, D)),
            full((1, D)),
            full((D, 128)),
            full((1, 128)),
        ],
        out_specs=pl.BlockSpec((BLK, 128), lambda i: (i, 0)),
        out_shape=jax.ShapeDtypeStruct((N, 128), jnp.float32),
    )(batch3d, x, aggp, aggp, read, Wc, Wa, Wr, b2, Wlp, blp)


def kernel(x, edge_index, batch, Wc0, Wa0, Wr0, b0, Wc1, Wa1, Wr1, b1, Wl, bl):
    N, D = x.shape
    C = Wl.shape[1]
    E = edge_index.shape[1]
    n_chunks = E // (NUM_W * EDGE_CHUNK)
    src3 = edge_index[0].reshape(NUM_W, 2, n_chunks // 2, EDGE_CHUNK)
    dst3 = edge_index[1].reshape(NUM_W, 2, n_chunks // 2, EDGE_CHUNK)
    batch3d = batch.reshape(N // BLK, 1, BLK)
    b0r = b0.reshape(1, D)
    b1r = b1.reshape(1, D)
    Wlp = jnp.zeros((D, 128), jnp.float32).at[:, :C].set(Wl)
    blp = jnp.zeros((1, 128), jnp.float32).at[0, :C].set(bl)

    read0 = _tc_read(x, batch3d)
    aggp0 = _sc_agg(x, src3, dst3)
    h0, read1 = _tc_combine_mid(x, aggp0, batch3d, read0, Wc0, Wa0, Wr0, b0r)
    aggp1 = _sc_agg(h0, src3, dst3)
    outp = _tc_combine_final(h0, aggp1, batch3d, read1, Wc1, Wa1, Wr1, b1r,
                             Wlp, blp)
    return outp[:, :C]
